# baseline (speedup 1.0000x reference)
import functools

import jax
import jax.numpy as jnp
from jax.experimental import pallas as pl
from jax.experimental.pallas import tpu as pltpu
from jax.experimental.pallas import tpu_sc as plsc

_B = 1024
_D = 1024
_N = 16384
_TEMP_INV = 20.0
_LMAX = 20.0
_SUB = 1024
_NSUB = 2
_NT = _SUB * _NSUB
_TILES = _N // _NT


def _gather_target_rows(features, targets):
    info = plsc.get_sparse_core_info()
    nw = info.num_cores * info.num_subcores
    bpw = _B // nw
    mesh = plsc.VectorSubcoreMesh(core_axis_name="c", subcore_axis_name="s")

    @functools.partial(
        pl.kernel, mesh=mesh,
        out_type=jax.ShapeDtypeStruct((_B, _D), jnp.float32),
        scratch_types=[
            pltpu.VMEM((bpw,), jnp.int32),
            pltpu.VMEM((bpw, _D), jnp.float32),
            pltpu.SemaphoreType.DMA,
        ],
    )
    def gather_kernel(table_hbm, idx_hbm, out_hbm, idx_v, rows_v, sem):
        wid = jax.lax.axis_index("s") * info.num_cores + jax.lax.axis_index("c")
        base = wid * bpw
        pltpu.sync_copy(idx_hbm.at[pl.ds(base, bpw)], idx_v)
        pltpu.async_copy(table_hbm.at[idx_v], rows_v, sem).wait()
        pltpu.sync_copy(rows_v, out_hbm.at[pl.ds(base, bpw)])

    return gather_kernel(features, targets)


def _loss_body(x_ref, f_ref, g_ref, o_ref, xn_ref, s_ref):
    i = pl.program_id(0)

    @pl.when(i == 0)
    def _init():
        x = x_ref[...]
        nrm = jnp.maximum(
            jnp.sqrt(jnp.sum(x * x, axis=1, keepdims=True)), 1e-12)
        xn_ref[...] = (x / nrm).astype(jnp.bfloat16)
        s_ref[...] = jnp.zeros((_B, 1), jnp.float32)

    s_acc = jnp.zeros((_B, 1), jnp.float32)
    for j in range(_NSUB):
        l = jax.lax.dot_general(
            xn_ref[...], f_ref[j * _SUB:(j + 1) * _SUB, :],
            (((1,), (1,)), ((), ())),
            preferred_element_type=jnp.float32,
            precision=jax.lax.Precision.DEFAULT)
        s_acc += jnp.sum(jnp.exp(l * _TEMP_INV - _LMAX), axis=1,
                         keepdims=True)
    s_ref[...] += s_acc

    @pl.when(i == _TILES - 1)
    def _fin():
        logz_sum = jnp.sum(_LMAX + jnp.log(s_ref[...]))
        tgt_sum = _TEMP_INV * jnp.sum(
            xn_ref[...].astype(jnp.float32) * g_ref[...])
        o_ref[...] = ((logz_sum - tgt_sum) * (1.0 / _B)).reshape(1, 1)


@functools.partial(jax.jit, static_argnames=())
def kernel(inputs, targets, features):
    g = _gather_target_rows(features, targets.astype(jnp.int32))
    out = pl.pallas_call(
        _loss_body,
        grid=(_TILES,),
        in_specs=[
            pl.BlockSpec((_B, _D), lambda i: (0, 0)),
            pl.BlockSpec((_NT, _D), lambda i: (i, 0)),
            pl.BlockSpec((_B, _D), lambda i: (0, 0)),
        ],
        out_specs=pl.BlockSpec((1, 1), lambda i: (0, 0)),
        out_shape=jax.ShapeDtypeStruct((1, 1), jnp.float32),
        scratch_shapes=[
            pltpu.VMEM((_B, _D), jnp.bfloat16),
            pltpu.VMEM((_B, 1), jnp.float32),
        ],
    )(inputs, features, g)
    return out[0, 0]


# speedup vs baseline: 1.0640x; 1.0640x over previous
import functools

import jax
import jax.numpy as jnp
from jax.experimental import pallas as pl
from jax.experimental.pallas import tpu as pltpu
from jax.experimental.pallas import tpu_sc as plsc

_B = 1024
_D = 1024
_N = 16384
_TEMP_INV = 20.0
_LMAX = 20.0
_SUB = 1024
_NSUB = 2
_NT = _SUB * _NSUB
_TILES = _N // _NT


def _gather_target_rows(features, targets):
    info = plsc.get_sparse_core_info()
    nw = info.num_cores * info.num_subcores
    bpw = _B // nw
    mesh = plsc.VectorSubcoreMesh(core_axis_name="c", subcore_axis_name="s")

    @functools.partial(
        pl.kernel, mesh=mesh,
        out_type=jax.ShapeDtypeStruct((_B, _D), jnp.float32),
        scratch_types=[
            pltpu.VMEM((bpw,), jnp.int32),
            pltpu.VMEM((bpw, _D), jnp.float32),
            pltpu.SemaphoreType.DMA,
        ],
    )
    def gather_kernel(table_hbm, idx_hbm, out_hbm, idx_v, rows_v, sem):
        wid = jax.lax.axis_index("s") * info.num_cores + jax.lax.axis_index("c")
        base = wid * bpw
        pltpu.sync_copy(idx_hbm.at[pl.ds(base, bpw)], idx_v)
        pltpu.async_copy(table_hbm.at[idx_v], rows_v, sem).wait()
        pltpu.sync_copy(rows_v, out_hbm.at[pl.ds(base, bpw)])

    return gather_kernel(features, targets)


def _main_body(x_ref, f_ref, s_out_ref, xn_out_ref, xn_ref, s_ref):
    i = pl.program_id(0)

    @pl.when(i == 0)
    def _init():
        x = x_ref[...]
        nrm = jnp.maximum(
            jnp.sqrt(jnp.sum(x * x, axis=1, keepdims=True)), 1e-12)
        xn_ref[...] = (x / nrm).astype(jnp.bfloat16)
        xn_out_ref[...] = xn_ref[...]
        s_ref[...] = jnp.zeros((_B, 1), jnp.float32)

    s_acc = jnp.zeros((_B, 1), jnp.float32)
    for j in range(_NSUB):
        l = jax.lax.dot_general(
            xn_ref[...], f_ref[j * _SUB:(j + 1) * _SUB, :],
            (((1,), (1,)), ((), ())),
            preferred_element_type=jnp.float32,
            precision=jax.lax.Precision.DEFAULT)
        s_acc += jnp.sum(jnp.exp(l * _TEMP_INV - _LMAX), axis=1,
                         keepdims=True)
    s_ref[...] += s_acc

    @pl.when(i == _TILES - 1)
    def _fin():
        s_out_ref[...] = s_ref[...]


def _combine_body(s_ref, xn_ref, g_ref, o_ref):
    logz_sum = jnp.sum(_LMAX + jnp.log(s_ref[...]))
    tgt_sum = _TEMP_INV * jnp.sum(xn_ref[...].astype(jnp.float32) * g_ref[...])
    o_ref[...] = ((logz_sum - tgt_sum) * (1.0 / _B)).reshape(1, 1)


@functools.partial(jax.jit, static_argnames=())
def kernel(inputs, targets, features):
    g = _gather_target_rows(features, targets.astype(jnp.int32))
    s, xn = pl.pallas_call(
        _main_body,
        grid=(_TILES,),
        in_specs=[
            pl.BlockSpec((_B, _D), lambda i: (0, 0)),
            pl.BlockSpec((_NT, _D), lambda i: (i, 0)),
        ],
        out_specs=[
            pl.BlockSpec((_B, 1), lambda i: (0, 0)),
            pl.BlockSpec((_B, _D), lambda i: (0, 0)),
        ],
        out_shape=[
            jax.ShapeDtypeStruct((_B, 1), jnp.float32),
            jax.ShapeDtypeStruct((_B, _D), jnp.bfloat16),
        ],
        scratch_shapes=[
            pltpu.VMEM((_B, _D), jnp.bfloat16),
            pltpu.VMEM((_B, 1), jnp.float32),
        ],
    )(inputs, features)
    out = pl.pallas_call(
        _combine_body,
        out_specs=pl.BlockSpec((1, 1), lambda: (0, 0)),
        out_shape=jax.ShapeDtypeStruct((1, 1), jnp.float32),
    )(s, xn, g)
    return out[0, 0]


# speedup vs baseline: 1.4309x; 1.3449x over previous
import functools

import jax
import jax.numpy as jnp
from jax.experimental import pallas as pl
from jax.experimental.pallas import tpu as pltpu

_B = 1024
_D = 1024
_N = 16384
_TEMP_INV = 20.0
_LMAX = 20.0
_SUB = 1024
_NSUB = 2
_NT = _SUB * _NSUB
_TILES = _N // _NT


def _loss_body(x_ref, t_ref, f_ref, o_ref, xn_ref, s_ref, te_ref):
    i = pl.program_id(0)

    @pl.when(i == 0)
    def _init():
        x = x_ref[...]
        nrm = jnp.maximum(
            jnp.sqrt(jnp.sum(x * x, axis=1, keepdims=True)), 1e-12)
        xn_ref[...] = (x / nrm).astype(jnp.bfloat16)
        s_ref[...] = jnp.zeros((1, _B), jnp.float32)
        te_ref[...] = jnp.zeros((1, _B), jnp.float32)

    s_acc = jnp.zeros((1, _B), jnp.float32)
    te_acc = jnp.zeros((1, _B), jnp.float32)
    for j in range(_NSUB):
        lt = jax.lax.dot_general(
            f_ref[j * _SUB:(j + 1) * _SUB, :], xn_ref[...],
            (((1,), (1,)), ((), ())),
            preferred_element_type=jnp.float32,
            precision=jax.lax.Precision.DEFAULT)
        e = jnp.exp(lt * _TEMP_INV - _LMAX)
        s_acc += jnp.sum(e, axis=0, keepdims=True)
        rows = (i * _NT + j * _SUB
                + jax.lax.broadcasted_iota(jnp.int32, (_SUB, _B), 0))
        hit = rows == t_ref[...]
        te_acc += jnp.sum(jnp.where(hit, e, 0.0), axis=0, keepdims=True)
    s_ref[...] += s_acc
    te_ref[...] += te_acc

    @pl.when(i == _TILES - 1)
    def _fin():
        loss = jnp.log(s_ref[...]) - jnp.log(te_ref[...])
        o_ref[...] = jnp.sum(loss, keepdims=True) * (1.0 / _B)


@functools.partial(jax.jit, static_argnames=())
def kernel(inputs, targets, features):
    out = pl.pallas_call(
        _loss_body,
        grid=(_TILES,),
        in_specs=[
            pl.BlockSpec((_B, _D), lambda i: (0, 0)),
            pl.BlockSpec((1, _B), lambda i: (0, 0)),
            pl.BlockSpec((_NT, _D), lambda i: (i, 0)),
        ],
        out_specs=pl.BlockSpec((1, 1), lambda i: (0, 0)),
        out_shape=jax.ShapeDtypeStruct((1, 1), jnp.float32),
        scratch_shapes=[
            pltpu.VMEM((_B, _D), jnp.bfloat16),
            pltpu.VMEM((1, _B), jnp.float32),
            pltpu.VMEM((1, _B), jnp.float32),
        ],
    )(inputs, targets.astype(jnp.int32).reshape(1, _B), features)
    return out[0, 0]
